# baseline (speedup 1.0000x reference)
import functools

import jax
import jax.numpy as jnp
from jax.experimental import pallas as pl

N = 10000
E = 160000
D_IN = 256
D_H = 1024
L = 4

_BLK = 1000


def _tc_layer_body(agg_ref, w_ref, b_ref, g_ref, be_ref, dinv_ref, out_ref):
    z = jnp.dot(agg_ref[...], w_ref[...], preferred_element_type=jnp.float32)
    z = z + b_ref[...]
    mu = jnp.mean(z, axis=-1, keepdims=True)
    var = jnp.mean((z - mu) ** 2, axis=-1, keepdims=True)
    h = (z - mu) * jax.lax.rsqrt(var + 1e-5) * g_ref[...] + be_ref[...]
    h = jnp.maximum(h, 0.0)
    out_ref[...] = h * dinv_ref[...]


def _tc_final_body(agg_ref, w_ref, b_ref, g_ref, be_ref, out_ref):
    z = jnp.dot(agg_ref[...], w_ref[...], preferred_element_type=jnp.float32)
    z = z + b_ref[...]
    mu = jnp.mean(z, axis=-1, keepdims=True)
    var = jnp.mean((z - mu) ** 2, axis=-1, keepdims=True)
    h = (z - mu) * jax.lax.rsqrt(var + 1e-5) * g_ref[...] + be_ref[...]
    h = jnp.maximum(h, 0.0)
    part = jnp.sum(h, axis=0, keepdims=True) * (1.0 / N)

    @pl.when(pl.program_id(0) == 0)
    def _():
        out_ref[...] = jnp.zeros_like(out_ref)

    out_ref[...] += part


def _tc_layer(agg, w, b, g, be, dinv_col):
    din = agg.shape[1]
    return pl.pallas_call(
        _tc_layer_body,
        grid=(N // _BLK,),
        in_specs=[
            pl.BlockSpec((_BLK, din), lambda i: (i, 0)),
            pl.BlockSpec((din, D_H), lambda i: (0, 0)),
            pl.BlockSpec((1, D_H), lambda i: (0, 0)),
            pl.BlockSpec((1, D_H), lambda i: (0, 0)),
            pl.BlockSpec((1, D_H), lambda i: (0, 0)),
            pl.BlockSpec((_BLK, 1), lambda i: (i, 0)),
        ],
        out_specs=pl.BlockSpec((_BLK, D_H), lambda i: (i, 0)),
        out_shape=jax.ShapeDtypeStruct((N, D_H), jnp.float32),
    )(agg, w, b.reshape(1, -1), g.reshape(1, -1), be.reshape(1, -1), dinv_col)


def _tc_final(agg, w, b, g, be):
    din = agg.shape[1]
    return pl.pallas_call(
        _tc_final_body,
        grid=(N // _BLK,),
        in_specs=[
            pl.BlockSpec((_BLK, din), lambda i: (i, 0)),
            pl.BlockSpec((din, D_H), lambda i: (0, 0)),
            pl.BlockSpec((1, D_H), lambda i: (0, 0)),
            pl.BlockSpec((1, D_H), lambda i: (0, 0)),
            pl.BlockSpec((1, D_H), lambda i: (0, 0)),
        ],
        out_specs=pl.BlockSpec((1, D_H), lambda i: (0, 0)),
        out_shape=jax.ShapeDtypeStruct((1, D_H), jnp.float32),
    )(agg, w, b.reshape(1, -1), g.reshape(1, -1), be.reshape(1, -1))


def _prescale_body(x_ref, dinv_ref, out_ref):
    out_ref[...] = x_ref[...] * dinv_ref[...]


def _prescale(x, dinv_col):
    return pl.pallas_call(
        _prescale_body,
        grid=(N // _BLK,),
        in_specs=[
            pl.BlockSpec((_BLK, D_IN), lambda i: (i, 0)),
            pl.BlockSpec((_BLK, 1), lambda i: (i, 0)),
        ],
        out_specs=pl.BlockSpec((_BLK, D_IN), lambda i: (i, 0)),
        out_shape=jax.ShapeDtypeStruct((N, D_IN), jnp.float32),
    )(x, dinv_col)


def kernel(x, edge_index, W0, b0, g0, be0, W1, b1, g1, be1, W2, b2, g2, be2,
           W3, b3, g3, be3):
    loop = jnp.arange(N, dtype=edge_index.dtype)
    src_full = jnp.concatenate([edge_index[0], loop])
    dst_full = jnp.concatenate([edge_index[1], loop])
    order = jnp.argsort(dst_full)
    dst_s = dst_full[order]
    src_s = src_full[order]
    row_ptr = jnp.searchsorted(dst_s, jnp.arange(N + 1, dtype=jnp.int32))
    deg = (row_ptr[1:] - row_ptr[:-1]).astype(jnp.float32)
    dinv = jax.lax.rsqrt(deg)
    dinv_col = dinv.reshape(N, 1)

    Ws = [W0, W1, W2, W3]
    bs = [b0, b1, b2, b3]
    gs = [g0, g1, g2, g3]
    bes = [be0, be1, be2, be3]

    hs = _prescale(x, dinv_col)
    for i in range(L):
        gathered = hs[src_s]
        seg = jax.ops.segment_sum(gathered, dst_s, num_segments=N,
                                  indices_are_sorted=True)
        agg = seg * dinv_col
        if i < L - 1:
            hs = _tc_layer(agg, Ws[i], bs[i], gs[i], bes[i], dinv_col)
        else:
            out = _tc_final(agg, Ws[i], bs[i], gs[i], bes[i])
    return out


# speedup vs baseline: 1.4151x; 1.4151x over previous
import functools

import jax
import jax.numpy as jnp
from jax import lax
from jax.experimental import pallas as pl
from jax.experimental.pallas import tpu as pltpu
from jax.experimental.pallas import tpu_sc as plsc

N = 10000
E = 160000
D_IN = 256
D_H = 1024
L = 4

NW = 32
V_PER = 320
K = 16
EP = E + N
E_PAD = EP + 128

_BLK = 1000


def _tc_layer_body(last, agg_ref, w_ref, b_ref, g_ref, be_ref, dinv_ref,
                   out_ref):
    a = agg_ref[...] * dinv_ref[...]
    z = jnp.dot(a, w_ref[...], preferred_element_type=jnp.float32)
    z = z + b_ref[...]
    mu = jnp.mean(z, axis=-1, keepdims=True)
    var = jnp.mean((z - mu) ** 2, axis=-1, keepdims=True)
    h = (z - mu) * lax.rsqrt(var + 1e-5) * g_ref[...] + be_ref[...]
    h = jnp.maximum(h, 0.0)
    if last:
        part = jnp.sum(h, axis=0, keepdims=True) * (1.0 / N)

        @pl.when(pl.program_id(0) == 0)
        def _():
            out_ref[...] = jnp.zeros_like(out_ref)

        out_ref[...] += part
    else:
        out_ref[...] = h * dinv_ref[...]


def _tc_layer(agg, w, b, g, be, dinv_col, last):
    din = agg.shape[1]
    if last:
        out_spec = pl.BlockSpec((1, D_H), lambda i: (0, 0))
        out_shape = jax.ShapeDtypeStruct((1, D_H), jnp.float32)
    else:
        out_spec = pl.BlockSpec((_BLK, D_H), lambda i: (i, 0))
        out_shape = jax.ShapeDtypeStruct((N, D_H), jnp.float32)
    return pl.pallas_call(
        functools.partial(_tc_layer_body, last),
        grid=(N // _BLK,),
        in_specs=[
            pl.BlockSpec((_BLK, din), lambda i: (i, 0)),
            pl.BlockSpec((din, D_H), lambda i: (0, 0)),
            pl.BlockSpec((1, D_H), lambda i: (0, 0)),
            pl.BlockSpec((1, D_H), lambda i: (0, 0)),
            pl.BlockSpec((1, D_H), lambda i: (0, 0)),
            pl.BlockSpec((_BLK, 1), lambda i: (i, 0)),
        ],
        out_specs=out_spec,
        out_shape=out_shape,
    )(agg, w, b.reshape(1, -1), g.reshape(1, -1), be.reshape(1, -1), dinv_col)


def _prescale_body(x_ref, dinv_ref, out_ref):
    out_ref[...] = x_ref[...] * dinv_ref[...]


def _prescale(x, dinv_col):
    return pl.pallas_call(
        _prescale_body,
        grid=(N // _BLK,),
        in_specs=[
            pl.BlockSpec((_BLK, D_IN), lambda i: (i, 0)),
            pl.BlockSpec((_BLK, 1), lambda i: (i, 0)),
        ],
        out_specs=pl.BlockSpec((_BLK, D_IN), lambda i: (i, 0)),
        out_shape=jax.ShapeDtypeStruct((N, D_IN), jnp.float32),
    )(x, dinv_col)


def _ext_i32(vec, lane, j):
    return jnp.sum(jnp.where(lane == j, vec, jnp.zeros_like(vec)))


def _agg_body(D, hs, srcm, flagm, tmeta, out,
              meta_buf, idx_ring, flag_ring, rows, acc, staging,
              gsem, msem):
    D16 = D // 16
    wid = lax.axis_index("s") * 2 + lax.axis_index("c")
    v0 = wid * V_PER
    lane = lax.broadcasted_iota(jnp.int32, (16,), 0)

    pltpu.sync_copy(tmeta.at[wid], meta_buf)
    mv = meta_buf[...]
    ae0 = _ext_i32(mv, lane, 0)
    nch = _ext_i32(mv, lane, 1)
    e0 = _ext_i32(mv, lane, 2)
    e1 = _ext_i32(mv, lane, 3)

    zero16 = jnp.zeros((16,), jnp.float32)

    def _zero_acc(d, carry):
        acc[d, :] = zero16
        return carry

    lax.fori_loop(0, D16, _zero_acc, 0)

    def _meta_fire(c):
        slot = lax.rem(c, 3)
        off = pl.multiple_of(ae0 + c * K, K)
        pltpu.make_async_copy(srcm.at[pl.ds(off, K)], idx_ring.at[slot],
                              msem.at[slot]).start()
        pltpu.make_async_copy(flagm.at[pl.ds(off, K)], flag_ring.at[slot],
                              msem.at[slot]).start()

    def _meta_wait(c):
        slot = lax.rem(c, 3)
        off = pl.multiple_of(ae0, K)
        pltpu.make_async_copy(srcm.at[pl.ds(off, K)], idx_ring.at[slot],
                              msem.at[slot]).wait()
        pltpu.make_async_copy(flagm.at[pl.ds(off, K)], flag_ring.at[slot],
                              msem.at[slot]).wait()

    def _gather_start(c):
        b = lax.rem(c, 2)
        pltpu.make_async_copy(hs.at[idx_ring.at[lax.rem(c, 3)]],
                              rows.at[pl.ds(b * K, K)], gsem.at[b]).start()

    def _gather_wait(c):
        b = lax.rem(c, 2)
        pltpu.make_async_copy(hs.at[idx_ring.at[lax.rem(c, 3)]],
                              rows.at[pl.ds(b * K, K)], gsem.at[b]).wait()

    _meta_fire(0)
    _meta_fire(1)
    _meta_wait(0)
    _gather_start(0)

    def chunk_body(c, flush_cnt):
        b = lax.rem(c, 2)
        _meta_fire(c + 2)
        _meta_wait(c + 1)
        _gather_start(c + 1)
        _gather_wait(c)

        flags = flag_ring[lax.rem(c, 3), :]
        gev = ae0 + c * K + lane
        act = (gev >= e0) & (gev < e1)
        wb = (flags > 0) & act
        wvi = jnp.where(wb, 1, 0).astype(jnp.int32)
        cum = plsc.cumsum(wvi)
        posv = flush_cnt + cum - wvi
        addrv = jnp.where(wb, lax.rem(posv, 32), 32)
        nfl = _ext_i32(cum, lane, 15)

        addr = [_ext_i32(addrv, lane, j) for j in range(K)]
        endf = [_ext_i32(flags, lane, j) > 0 for j in range(K)]

        def dbody(d, carry):
            sl = pl.ds(d * 16, 16)
            r = acc[d, :]
            for j in range(K):
                r = r + rows[b * K + j, sl]
                staging[addr[j], sl] = r
                r = jnp.where(endf[j], 0.0, r)
            acc[d, :] = r
            return carry

        lax.fori_loop(0, D16, dbody, 0)

        new_cnt = flush_cnt + nfl

        @pl.when(new_cnt // 16 > flush_cnt // 16)
        def _():
            blk = flush_cnt // 16
            pltpu.sync_copy(staging.at[pl.ds(lax.rem(blk, 2) * 16, 16)],
                            out.at[pl.ds(v0 + blk * 16, 16)])

        return new_cnt

    lax.fori_loop(0, nch, chunk_body, 0)

    _gather_wait(nch)
    _meta_wait(nch + 1)


def _make_agg(D):
    mesh = plsc.VectorSubcoreMesh(core_axis_name="c", subcore_axis_name="s")
    D16 = D // 16
    return pl.kernel(
        functools.partial(_agg_body, D),
        out_type=jax.ShapeDtypeStruct((N, D), jnp.float32),
        mesh=mesh,
        compiler_params=pltpu.CompilerParams(needs_layout_passes=False),
        scratch_types=[
            pltpu.VMEM((16,), jnp.int32),
            pltpu.VMEM((3, K), jnp.int32),
            pltpu.VMEM((3, K), jnp.float32),
            pltpu.VMEM((2 * K, D), jnp.float32),
            pltpu.VMEM((D16, 16), jnp.float32),
            pltpu.VMEM((33, D), jnp.float32),
            pltpu.SemaphoreType.DMA((2,)),
            pltpu.SemaphoreType.DMA((3,)),
        ],
    )


_agg_256 = _make_agg(D_IN)
_agg_1024 = _make_agg(D_H)


def kernel(x, edge_index, W0, b0, g0, be0, W1, b1, g1, be1, W2, b2, g2, be2,
           W3, b3, g3, be3):
    loop = jnp.arange(N, dtype=edge_index.dtype)
    src_full = jnp.concatenate([edge_index[0], loop])
    dst_full = jnp.concatenate([edge_index[1], loop])
    order = jnp.argsort(dst_full)
    dst_s = dst_full[order]
    src_s = src_full[order]
    row_ptr = jnp.searchsorted(dst_s, jnp.arange(N + 1, dtype=jnp.int32))
    row_ptr = row_ptr.astype(jnp.int32)
    deg = (row_ptr[1:] - row_ptr[:-1]).astype(jnp.float32)
    dinv = lax.rsqrt(deg)
    dinv_col = dinv.reshape(N, 1)

    seg_end = jnp.concatenate(
        [(dst_s[1:] != dst_s[:-1]), jnp.ones((1,), bool)]).astype(jnp.float32)
    src_pad = jnp.concatenate(
        [src_s.astype(jnp.int32), jnp.zeros((E_PAD - EP,), jnp.int32)])
    flag_pad = jnp.concatenate([seg_end, jnp.zeros((E_PAD - EP,), jnp.float32)])

    v0s = jnp.arange(NW, dtype=jnp.int32) * V_PER
    v1s = jnp.minimum(v0s + V_PER, N)
    e0s = row_ptr[v0s]
    e1s = row_ptr[v1s]
    ae0s = (e0s // K) * K
    nchs = (e1s - ae0s + K - 1) // K
    tmeta = jnp.concatenate(
        [ae0s[:, None], nchs[:, None], e0s[:, None], e1s[:, None],
         jnp.zeros((NW, 12), jnp.int32)], axis=1)

    Ws = [W0, W1, W2, W3]
    bs = [b0, b1, b2, b3]
    gs = [g0, g1, g2, g3]
    bes = [be0, be1, be2, be3]

    hs = _prescale(x, dinv_col)
    for i in range(L):
        aggf = _agg_256 if i == 0 else _agg_1024
        agg = aggf(hs, src_pad, flag_pad, tmeta)
        hs = _tc_layer(agg, Ws[i], bs[i], gs[i], bes[i], dinv_col,
                       last=(i == L - 1))
    return hs


# speedup vs baseline: 5.8965x; 4.1668x over previous
import functools

import jax
import jax.numpy as jnp
from jax import lax
from jax.experimental import pallas as pl
from jax.experimental.pallas import tpu as pltpu
from jax.experimental.pallas import tpu_sc as plsc

N = 10000
E = 160000
D_IN = 256
D_H = 1024
L = 4

NW = 32
V_PER = 320
K = 16
EP = E + N
E_PAD = EP + 128

_BLK = 1000


def _tc_layer_body(last, agg_ref, w_ref, b_ref, g_ref, be_ref, dinv_ref,
                   out_ref):
    a = agg_ref[...] * dinv_ref[...]
    z = jnp.dot(a, w_ref[...], preferred_element_type=jnp.float32)
    z = z + b_ref[...]
    mu = jnp.mean(z, axis=-1, keepdims=True)
    var = jnp.mean((z - mu) ** 2, axis=-1, keepdims=True)
    h = (z - mu) * lax.rsqrt(var + 1e-5) * g_ref[...] + be_ref[...]
    h = jnp.maximum(h, 0.0)
    if last:
        part = jnp.sum(h, axis=0, keepdims=True) * (1.0 / N)

        @pl.when(pl.program_id(0) == 0)
        def _():
            out_ref[...] = jnp.zeros_like(out_ref)

        out_ref[...] += part
    else:
        out_ref[...] = h * dinv_ref[...]


def _tc_layer(agg, w, b, g, be, dinv_col, last):
    din = agg.shape[1]
    if last:
        out_spec = pl.BlockSpec((1, D_H), lambda i: (0, 0))
        out_shape = jax.ShapeDtypeStruct((1, D_H), jnp.float32)
    else:
        out_spec = pl.BlockSpec((_BLK, D_H), lambda i: (i, 0))
        out_shape = jax.ShapeDtypeStruct((N, D_H), jnp.float32)
    return pl.pallas_call(
        functools.partial(_tc_layer_body, last),
        grid=(N // _BLK,),
        in_specs=[
            pl.BlockSpec((_BLK, din), lambda i: (i, 0)),
            pl.BlockSpec((din, D_H), lambda i: (0, 0)),
            pl.BlockSpec((1, D_H), lambda i: (0, 0)),
            pl.BlockSpec((1, D_H), lambda i: (0, 0)),
            pl.BlockSpec((1, D_H), lambda i: (0, 0)),
            pl.BlockSpec((_BLK, 1), lambda i: (i, 0)),
        ],
        out_specs=out_spec,
        out_shape=out_shape,
    )(agg, w, b.reshape(1, -1), g.reshape(1, -1), be.reshape(1, -1), dinv_col)


def _prescale_body(x_ref, dinv_ref, out_ref):
    out_ref[...] = x_ref[...] * dinv_ref[...]


def _prescale(x, dinv_col):
    return pl.pallas_call(
        _prescale_body,
        grid=(N // _BLK,),
        in_specs=[
            pl.BlockSpec((_BLK, D_IN), lambda i: (i, 0)),
            pl.BlockSpec((_BLK, 1), lambda i: (i, 0)),
        ],
        out_specs=pl.BlockSpec((_BLK, D_IN), lambda i: (i, 0)),
        out_shape=jax.ShapeDtypeStruct((N, D_IN), jnp.float32),
    )(x, dinv_col)


def _ext_i32(vec, lane, j):
    return jnp.sum(jnp.where(lane == j, vec, jnp.zeros_like(vec)))


def _agg_body(D, hs, srcm, flagm, tmeta, out,
              meta_buf, idx_ring, flag_ring, rows, acc, staging,
              gsem, msem):
    D16 = D // 16
    wid = lax.axis_index("s") * 2 + lax.axis_index("c")
    v0 = wid * V_PER
    lane = lax.broadcasted_iota(jnp.int32, (16,), 0)

    pltpu.sync_copy(tmeta.at[wid], meta_buf)
    mv = meta_buf[...]
    ae0 = _ext_i32(mv, lane, 0)
    nch = _ext_i32(mv, lane, 1)
    e0 = _ext_i32(mv, lane, 2)
    e1 = _ext_i32(mv, lane, 3)

    zero16 = jnp.zeros((16,), jnp.float32)

    def _zero_acc(d, carry):
        acc[d, :] = zero16
        return carry

    lax.fori_loop(0, D16, _zero_acc, 0)

    def _meta_fire(c):
        slot = lax.rem(c, 3)
        off = pl.multiple_of(ae0 + c * K, K)
        pltpu.make_async_copy(srcm.at[pl.ds(off, K)], idx_ring.at[slot],
                              msem.at[slot]).start()
        pltpu.make_async_copy(flagm.at[pl.ds(off, K)], flag_ring.at[slot],
                              msem.at[slot]).start()

    def _meta_wait(c):
        slot = lax.rem(c, 3)
        off = pl.multiple_of(ae0, K)
        pltpu.make_async_copy(srcm.at[pl.ds(off, K)], idx_ring.at[slot],
                              msem.at[slot]).wait()
        pltpu.make_async_copy(flagm.at[pl.ds(off, K)], flag_ring.at[slot],
                              msem.at[slot]).wait()

    def _gather_start(c):
        b = lax.rem(c, 2)
        pltpu.make_async_copy(hs.at[idx_ring.at[lax.rem(c, 3)]],
                              rows.at[pl.ds(b * K, K)], gsem.at[b]).start()

    def _gather_wait(c):
        b = lax.rem(c, 2)
        pltpu.make_async_copy(hs.at[idx_ring.at[lax.rem(c, 3)]],
                              rows.at[pl.ds(b * K, K)], gsem.at[b]).wait()

    _meta_fire(0)
    _meta_fire(1)
    _meta_wait(0)
    _gather_start(0)

    def chunk_body(c, flush_cnt):
        b = lax.rem(c, 2)
        _meta_fire(c + 2)
        _meta_wait(c + 1)
        _gather_start(c + 1)
        _gather_wait(c)

        flags = flag_ring[lax.rem(c, 3), :]
        gev = ae0 + c * K + lane
        act = (gev >= e0) & (gev < e1)
        wb = (flags > 0) & act
        wvi = jnp.where(wb, 1, 0).astype(jnp.int32)
        cum = plsc.cumsum(wvi)
        posv = flush_cnt + cum - wvi
        addrv = jnp.where(wb, lax.rem(posv, 32), 32)
        nfl = _ext_i32(cum, lane, 15)

        addr = [_ext_i32(addrv, lane, j) for j in range(K)]
        endf = [_ext_i32(flags, lane, j) > 0 for j in range(K)]

        @plsc.parallel_loop(0, D16, step=1, unroll=4)
        def _dbody(d):
            sl = pl.ds(d * 16, 16)
            r = acc[d, :]
            for j in range(K):
                r = r + rows[b * K + j, sl]
                staging[addr[j], sl] = r
                r = jnp.where(endf[j], 0.0, r)
            acc[d, :] = r

        new_cnt = flush_cnt + nfl

        @pl.when(new_cnt // 16 > flush_cnt // 16)
        def _():
            blk = flush_cnt // 16
            pltpu.sync_copy(staging.at[pl.ds(lax.rem(blk, 2) * 16, 16)],
                            out.at[pl.ds(v0 + blk * 16, 16)])

        return new_cnt

    lax.fori_loop(0, nch, chunk_body, 0)

    _gather_wait(nch)
    _meta_wait(nch + 1)


def _make_agg(D):
    mesh = plsc.VectorSubcoreMesh(core_axis_name="c", subcore_axis_name="s")
    D16 = D // 16
    return pl.kernel(
        functools.partial(_agg_body, D),
        out_type=jax.ShapeDtypeStruct((N, D), jnp.float32),
        mesh=mesh,
        compiler_params=pltpu.CompilerParams(needs_layout_passes=False),
        scratch_types=[
            pltpu.VMEM((16,), jnp.int32),
            pltpu.VMEM((3, K), jnp.int32),
            pltpu.VMEM((3, K), jnp.float32),
            pltpu.VMEM((2 * K, D), jnp.float32),
            pltpu.VMEM((D16, 16), jnp.float32),
            pltpu.VMEM((33, D), jnp.float32),
            pltpu.SemaphoreType.DMA((2,)),
            pltpu.SemaphoreType.DMA((3,)),
        ],
    )


_agg_256 = _make_agg(D_IN)
_agg_1024 = _make_agg(D_H)


def kernel(x, edge_index, W0, b0, g0, be0, W1, b1, g1, be1, W2, b2, g2, be2,
           W3, b3, g3, be3):
    loop = jnp.arange(N, dtype=edge_index.dtype)
    src_full = jnp.concatenate([edge_index[0], loop])
    dst_full = jnp.concatenate([edge_index[1], loop])
    packed = (dst_full.astype(jnp.uint32) << 14) | src_full.astype(jnp.uint32)
    packed_s = lax.sort(packed)
    src_s = (packed_s & jnp.uint32((1 << 14) - 1)).astype(jnp.int32)
    dst_s = (packed_s >> 14).astype(jnp.int32)
    row_ptr = jnp.searchsorted(dst_s, jnp.arange(N + 1, dtype=jnp.int32))
    row_ptr = row_ptr.astype(jnp.int32)
    deg = (row_ptr[1:] - row_ptr[:-1]).astype(jnp.float32)
    dinv = lax.rsqrt(deg)
    dinv_col = dinv.reshape(N, 1)

    seg_end = jnp.concatenate(
        [(dst_s[1:] != dst_s[:-1]), jnp.ones((1,), bool)]).astype(jnp.float32)
    src_pad = jnp.concatenate(
        [src_s.astype(jnp.int32), jnp.zeros((E_PAD - EP,), jnp.int32)])
    flag_pad = jnp.concatenate([seg_end, jnp.zeros((E_PAD - EP,), jnp.float32)])

    v0s = jnp.arange(NW, dtype=jnp.int32) * V_PER
    v1s = jnp.minimum(v0s + V_PER, N)
    e0s = row_ptr[v0s]
    e1s = row_ptr[v1s]
    ae0s = (e0s // K) * K
    nchs = (e1s - ae0s + K - 1) // K
    tmeta = jnp.concatenate(
        [ae0s[:, None], nchs[:, None], e0s[:, None], e1s[:, None],
         jnp.zeros((NW, 12), jnp.int32)], axis=1)

    Ws = [W0, W1, W2, W3]
    bs = [b0, b1, b2, b3]
    gs = [g0, g1, g2, g3]
    bes = [be0, be1, be2, be3]

    hs = _prescale(x, dinv_col)
    for i in range(L):
        aggf = _agg_256 if i == 0 else _agg_1024
        agg = aggf(hs, src_pad, flag_pad, tmeta)
        hs = _tc_layer(agg, Ws[i], bs[i], gs[i], bes[i], dinv_col,
                       last=(i == L - 1))
    return hs
